# baseline (speedup 1.0000x reference)
import jax
import jax.numpy as jnp
from jax import lax
from jax.experimental import pallas as pl
from jax.experimental.pallas import tpu as pltpu
from jax.experimental.pallas import tpu_sc as plsc

B = 16384
DSET = 1000
DOM = 100
M = 16
D = 128

NC = 2
NS = 16
NW = NC * NS
BPW = B // NW

ROW = M * D
SUB = ROW // 8

STAGE_ROWS = 512
STAGE_REM = 8 * DSET - 15 * STAGE_ROWS


def _sc_body(ds_hbm, dom_hbm, sys_hbm, domtab_hbm, out1_hbm, out2_hbm,
             ds_v, dom_v, sys_sp, sem_g, sem_w, sem_stage):
    sid = lax.axis_index("s")
    wid = sid * NC + lax.axis_index("c")
    base = wid * BPW

    @pl.when(sid < 15)
    def _():
        pltpu.async_copy(sys_hbm.at[pl.ds(sid * STAGE_ROWS, STAGE_ROWS)],
                         sys_sp.at[pl.ds(sid * STAGE_ROWS, STAGE_ROWS)],
                         sem_stage)

    @pl.when(sid == 15)
    def _():
        pltpu.async_copy(sys_hbm.at[pl.ds(15 * STAGE_ROWS, STAGE_REM)],
                         sys_sp.at[pl.ds(15 * STAGE_ROWS, STAGE_REM)],
                         sem_stage)

    pltpu.sync_copy(ds_hbm.at[pl.ds(base, BPW)], ds_v)
    pltpu.sync_copy(dom_hbm.at[pl.ds(base, BPW)], dom_v)

    def dom_dma(c, _):
        off = pl.multiple_of(c * 16, 8)
        vf = ds_v[pl.ds(off, 16)] * DOM + dom_v[pl.ds(off, 16)]
        for j in range(16):
            f = vf[j]
            pltpu.async_copy(
                domtab_hbm.at[pl.ds(pl.multiple_of(f * D, 8), D)],
                out2_hbm.at[pl.ds(
                    pl.multiple_of((base + c * 16 + j) * D, 8), D)],
                sem_g)
        return _

    lax.fori_loop(0, BPW // 16, dom_dma, 0)

    @pl.when(sid < 15)
    def _():
        pltpu.make_async_copy(
            sys_hbm.at[pl.ds(0, STAGE_ROWS)],
            sys_sp.at[pl.ds(0, STAGE_ROWS)], sem_stage).wait()

    @pl.when(sid == 15)
    def _():
        pltpu.make_async_copy(
            sys_hbm.at[pl.ds(0, STAGE_REM)],
            sys_sp.at[pl.ds(0, STAGE_REM)], sem_stage).wait()

    plsc.subcore_barrier()

    def row_dma(c, _):
        off = pl.multiple_of(c * 16, 8)
        vds = ds_v[pl.ds(off, 16)]
        for j in range(16):
            r = vds[j]
            pltpu.async_copy(
                sys_sp.at[pl.ds(pl.multiple_of(r * 8, 8), 8)],
                out1_hbm.at[pl.ds(
                    pl.multiple_of((base + c * 16 + j) * 8, 8), 8)],
                sem_w)
        return _

    lax.fori_loop(0, BPW // 16, row_dma, 0)

    def drain(i, _):
        pltpu.make_async_copy(
            sys_sp.at[pl.ds(0, 8)], out1_hbm.at[pl.ds(0, 8)], sem_w).wait()
        pltpu.make_async_copy(
            domtab_hbm.at[pl.ds(0, D)], out2_hbm.at[pl.ds(0, D)],
            sem_g).wait()
        return _

    lax.fori_loop(0, BPW, drain, 0)


@jax.jit
def _sc_call(dataset_id, domain_id, sys_flat, dom_flat):
    mesh = plsc.VectorSubcoreMesh(core_axis_name="c", subcore_axis_name="s",
                                  num_cores=NC, num_subcores=NS)
    return pl.kernel(
        _sc_body,
        out_type=(
            jax.ShapeDtypeStruct((8 * B, SUB), jnp.float32),
            jax.ShapeDtypeStruct((B * D,), jnp.float32),
        ),
        mesh=mesh,
        scratch_types=[
            pltpu.VMEM((BPW,), jnp.int32),
            pltpu.VMEM((BPW,), jnp.int32),
            pltpu.VMEM_SHARED((8 * DSET, SUB), jnp.float32),
            pltpu.SemaphoreType.DMA,
            pltpu.SemaphoreType.DMA,
            pltpu.SemaphoreType.DMA,
        ],
    )(dataset_id, domain_id, sys_flat, dom_flat)


def kernel(Dataset_id, Domain_id, system_prompts, domain_prompts,
           phys_dataset_emb, phys_domain_emb):
    del phys_dataset_emb, phys_domain_emb
    sys_flat = system_prompts.reshape(8 * DSET, SUB)
    dom_flat = domain_prompts.reshape(DSET * DOM * D)
    out1, out2 = _sc_call(Dataset_id, Domain_id, sys_flat, dom_flat)
    return out1.reshape(B, M, D), out2.reshape(B, D)


# speedup vs baseline: 3.2380x; 3.2380x over previous
import functools

import jax
import jax.numpy as jnp
from jax import lax
from jax.experimental import pallas as pl
from jax.experimental.pallas import tpu as pltpu
from jax.experimental.pallas import tpu_sc as plsc

B = 16384
DSET = 1000
DOM = 100
M = 16
D = 128


G = 256
NG = B // G


def _tc_body(ds_smem, table_ref, out_ref):
    g = pl.program_id(0)

    def body(j, carry):
        r = ds_smem[g * G + j]
        out_ref[j] = table_ref[r]
        return carry

    lax.fori_loop(0, G, body, 0, unroll=8)


@jax.jit
def _tc_call(dataset_id, sys4d):
    return pl.pallas_call(
        _tc_body,
        grid_spec=pltpu.PrefetchScalarGridSpec(
            num_scalar_prefetch=1,
            grid=(NG,),
            in_specs=[
                pl.BlockSpec((DSET, 2, 8, D), lambda g, ds: (0, 0, 0, 0)),
            ],
            out_specs=pl.BlockSpec((G, 2, 8, D), lambda g, ds: (g, 0, 0, 0)),
        ),
        out_shape=jax.ShapeDtypeStruct((B, 2, 8, D), jnp.float32),
    )(dataset_id, sys4d)


NC = 2
NS = 16
NW = NC * NS
BPW = B // NW
L = 16

C2 = 128
N2 = BPW // C2


def _sc_body(ds_hbm, dom_hbm, domtab_hbm, out2_hbm,
             ds_v, flat_v, buf2, sem_g, sem_w):
    wid = lax.axis_index("s") * NC + lax.axis_index("c")
    base = wid * BPW

    pltpu.sync_copy(ds_hbm.at[pl.ds(base, BPW)], ds_v)
    pltpu.sync_copy(dom_hbm.at[pl.ds(base, BPW)], flat_v)

    for i in range(BPW // L):
        sl = pl.ds(i * L, L)
        flat_v[sl] = ds_v[sl] * DOM + flat_v[sl]

    g = pltpu.async_copy(domtab_hbm.at[flat_v.at[pl.ds(0, C2)]],
                         buf2.at[0], sem_g)
    writes = []
    for c in range(N2):
        g.wait()
        if c + 1 < N2:
            g = pltpu.async_copy(
                domtab_hbm.at[flat_v.at[pl.ds((c + 1) * C2, C2)]],
                buf2.at[(c + 1) % 2], sem_g)
        if len(writes) == 2:
            writes.pop(0).wait()
        writes.append(pltpu.async_copy(
            buf2.at[c % 2], out2_hbm.at[pl.ds(base + c * C2, C2)], sem_w))
    for w in writes:
        w.wait()


@jax.jit
def _sc_call(dataset_id, domain_id, dom_flat):
    mesh = plsc.VectorSubcoreMesh(core_axis_name="c", subcore_axis_name="s",
                                  num_cores=NC, num_subcores=NS)
    return pl.kernel(
        _sc_body,
        out_type=jax.ShapeDtypeStruct((B, D), jnp.float32),
        mesh=mesh,
        scratch_types=[
            pltpu.VMEM((BPW,), jnp.int32),
            pltpu.VMEM((BPW,), jnp.int32),
            pltpu.VMEM((2, C2, D), jnp.float32),
            pltpu.SemaphoreType.DMA,
            pltpu.SemaphoreType.DMA,
        ],
    )(dataset_id, domain_id, dom_flat)


def kernel(Dataset_id, Domain_id, system_prompts, domain_prompts,
           phys_dataset_emb, phys_domain_emb):
    del phys_dataset_emb, phys_domain_emb
    sys4d = system_prompts.reshape(DSET, 2, 8, D)
    dom_flat = domain_prompts.reshape(DSET * DOM, D)
    out2 = _sc_call(Dataset_id, Domain_id, dom_flat)
    out1 = _tc_call(Dataset_id, sys4d)
    return out1.reshape(B, M, D), out2
